# baseline (speedup 1.0000x reference)
import functools

import jax
import jax.numpy as jnp
from jax import lax
from jax.experimental import pallas as pl
from jax.experimental.pallas import tpu as pltpu
from jax.experimental.pallas import tpu_sc as plsc

B, S, H = 4, 8192, 768
E, K = 64, 8
T = B * S
P = T * K
BLK = 256
NB = P // BLK + E
PADDED = NB * BLK
BT = 512

NC, NS = 2, 16
NW = NC * NS
ROWS_PER_W = PADDED // NW
GCH = 64
TPW = T // NW
CT = 8

_INTERPRET = False


def _router_body(x_ref, gw_ref, gb_ref, logits_ref, topw_ref, topi_ref):
    x = x_ref[...]
    logits = lax.dot_general(x, gw_ref[...], (((1,), (1,)), ((), ())),
                             preferred_element_type=jnp.float32) + gb_ref[...]
    logits_ref[...] = logits
    cur = logits
    iota = lax.broadcasted_iota(jnp.int32, (BT, E), 1)
    ws, ids = [], []
    for _ in range(K):
        m = jnp.max(cur, axis=1, keepdims=True)
        idx = jnp.min(jnp.where(cur == m, iota, E), axis=1, keepdims=True)
        ws.append(m)
        ids.append(idx)
        cur = jnp.where(iota == idx, -1e30, cur)
    l8 = jnp.concatenate(ws, axis=1)
    e8 = jnp.exp(l8 - l8[:, 0:1])
    topw_ref[...] = e8 / jnp.sum(e8, axis=1, keepdims=True)
    topi_ref[...] = jnp.concatenate(ids, axis=1)


def _router(hs, gate_w, gate_b2):
    return pl.pallas_call(
        _router_body,
        grid=(T // BT,),
        in_specs=[
            pl.BlockSpec((BT, H), lambda i: (i, 0)),
            pl.BlockSpec((E, H), lambda i: (0, 0)),
            pl.BlockSpec((1, E), lambda i: (0, 0)),
        ],
        out_specs=[
            pl.BlockSpec((BT, E), lambda i: (i, 0)),
            pl.BlockSpec((BT, K), lambda i: (i, 0)),
            pl.BlockSpec((BT, K), lambda i: (i, 0)),
        ],
        out_shape=[
            jax.ShapeDtypeStruct((T, E), jnp.float32),
            jax.ShapeDtypeStruct((T, K), jnp.float32),
            jax.ShapeDtypeStruct((T, K), jnp.int32),
        ],
        interpret=_INTERPRET,
    )(hs, gate_w, gate_b2)


def _gmm_body(beid_ref, x_ref, w_ref, b_ref, wp_ref, out_ref):
    del beid_ref
    x16 = x_ref[...].astype(jnp.bfloat16)
    w16 = w_ref[0].astype(jnp.bfloat16)
    acc = lax.dot_general(x16, w16, (((1,), (1,)), ((), ())),
                          preferred_element_type=jnp.float32)
    out_ref[...] = (acc + b_ref[0]) * wp_ref[...]


def _grouped_matmul(block_eid, xs, expert_w, expert_b, wp):
    grid_spec = pltpu.PrefetchScalarGridSpec(
        num_scalar_prefetch=1,
        grid=(NB,),
        in_specs=[
            pl.BlockSpec((BLK, H), lambda i, beid: (i, 0)),
            pl.BlockSpec((1, H, H), lambda i, beid: (beid[i], 0, 0)),
            pl.BlockSpec((1, 1, H), lambda i, beid: (beid[i], 0, 0)),
            pl.BlockSpec((BLK, 1), lambda i, beid: (i, 0)),
        ],
        out_specs=pl.BlockSpec((BLK, H), lambda i, beid: (i, 0)),
    )
    return pl.pallas_call(
        _gmm_body,
        grid_spec=grid_spec,
        out_shape=jax.ShapeDtypeStruct((PADDED, H), jnp.float32),
        interpret=_INTERPRET,
    )(block_eid, xs, expert_w, expert_b, wp)


def _make_sc_mesh():
    return plsc.VectorSubcoreMesh(core_axis_name="c", subcore_axis_name="s",
                                  num_cores=NC, num_subcores=NS)


def _sc_gather(hs, tok_at):
    @functools.partial(
        pl.kernel,
        out_type=jax.ShapeDtypeStruct((PADDED, H), jnp.float32),
        mesh=_make_sc_mesh(),
        scratch_types=[
            pltpu.VMEM((GCH,), jnp.int32),
            pltpu.VMEM((GCH,), jnp.int32),
            pltpu.VMEM((GCH, H), jnp.float32),
            pltpu.VMEM((GCH, H), jnp.float32),
            pltpu.SemaphoreType.DMA,
            pltpu.SemaphoreType.DMA,
        ],
    )
    def k(hs_hbm, tok_hbm, out_hbm, idx0, idx1, buf0, buf1, sem0, sem1):
        wid = lax.axis_index("s") * NC + lax.axis_index("c")
        base = wid * ROWS_PER_W

        def body(i, carry):
            c0 = base + i * 2 * GCH
            pltpu.sync_copy(tok_hbm.at[pl.ds(c0, GCH)], idx0)
            pltpu.sync_copy(tok_hbm.at[pl.ds(c0 + GCH, GCH)], idx1)
            cp0 = pltpu.async_copy(hs_hbm.at[idx0], buf0, sem0)
            cp1 = pltpu.async_copy(hs_hbm.at[idx1], buf1, sem1)
            cp0.wait()
            pltpu.sync_copy(buf0, out_hbm.at[pl.ds(c0, GCH)])
            cp1.wait()
            pltpu.sync_copy(buf1, out_hbm.at[pl.ds(c0 + GCH, GCH)])
            return carry

        lax.fori_loop(0, ROWS_PER_W // (2 * GCH), body, 0)

    return k(hs, tok_at)


def _sc_combine(out_rows, pos):
    @functools.partial(
        pl.kernel,
        out_type=jax.ShapeDtypeStruct((T, H), jnp.float32),
        mesh=_make_sc_mesh(),
        scratch_types=[
            pltpu.VMEM((CT * K,), jnp.int32),
            pltpu.VMEM((CT * K, H), jnp.float32),
            pltpu.VMEM((CT, H), jnp.float32),
            pltpu.SemaphoreType.DMA,
        ],
    )
    def k(rows_hbm, pos_hbm, out_hbm, idx_v, rows_v, acc_v, sem):
        wid = lax.axis_index("s") * NC + lax.axis_index("c")
        tbase = wid * TPW

        def body(i, carry):
            t0 = tbase + i * CT
            pltpu.sync_copy(pos_hbm.at[pl.ds(t0 * K, CT * K)], idx_v)
            pltpu.async_copy(rows_hbm.at[idx_v], rows_v, sem).wait()

            def jbody(j, c2):
                jj = pl.ds(pl.multiple_of(j * 16, 16), 16)
                for t in range(CT):
                    acc = rows_v[t * K, jj]
                    for r in range(1, K):
                        acc = acc + rows_v[t * K + r, jj]
                    acc_v[t, jj] = acc
                return c2

            lax.fori_loop(0, H // 16, jbody, 0)
            pltpu.sync_copy(acc_v, out_hbm.at[pl.ds(t0, CT)])
            return carry

        lax.fori_loop(0, TPW // CT, body, 0)

    return k(out_rows, pos)


def kernel(x, gate_w, gate_b, expert_w, expert_b):
    hs = x.reshape(T, H)
    logits, topw, topi = _router(hs, gate_w, gate_b.reshape(1, E))

    onehot = (topi[:, :, None] == jnp.arange(E, dtype=jnp.int32)[None, None, :])
    cnt = onehot.sum(axis=1).astype(jnp.int32)
    csum = jnp.cumsum(cnt, axis=0)
    counts = csum[-1]
    cexcl = (csum - cnt).astype(jnp.float32)
    rank = jnp.einsum("tke,te->tk", onehot.astype(jnp.float32), cexcl)

    padded_counts = ((counts + BLK - 1) // BLK) * BLK
    padded_end = jnp.cumsum(padded_counts).astype(jnp.int32)
    padded_off = (padded_end - padded_counts).astype(jnp.float32)
    off_tk = jnp.einsum("tke,e->tk", onehot.astype(jnp.float32), padded_off)
    dest = (rank + off_tk).astype(jnp.int32)

    block_eid = jnp.searchsorted(
        padded_end, jnp.arange(NB, dtype=jnp.int32) * BLK, side="right")
    block_eid = jnp.minimum(block_eid, E - 1).astype(jnp.int32)

    pos = dest.reshape(P)
    pair_at = jnp.full((PADDED,), -1, jnp.int32).at[pos].set(
        jnp.arange(P, dtype=jnp.int32))
    tok_at = jnp.where(pair_at >= 0, pair_at // K, 0).astype(jnp.int32)
    w_at = jnp.where(pair_at >= 0,
                     topw.reshape(P)[jnp.maximum(pair_at, 0)], 0.0)

    xs = _sc_gather(hs, tok_at)
    out_rows = _grouped_matmul(block_eid, xs, expert_w,
                               expert_b.reshape(E, 1, H),
                               w_at.reshape(PADDED, 1))
    final = _sc_combine(out_rows, pos)
    return final.reshape(B, S, H), logits


# speedup vs baseline: 1.7553x; 1.7553x over previous
import functools

import jax
import jax.numpy as jnp
from jax import lax
from jax.experimental import pallas as pl
from jax.experimental.pallas import tpu as pltpu
from jax.experimental.pallas import tpu_sc as plsc

B, S, H = 4, 8192, 768
E, K = 64, 8
T = B * S
P = T * K
BLK = 256
NB = P // BLK + E
PADDED = NB * BLK
BT = 512

NC, NS = 2, 16
NW = NC * NS
ROWS_PER_W = PADDED // NW
GCH = 64
TPW = T // NW
CT = 8

_INTERPRET = False


def _router_body(x_ref, gw_ref, gb_ref, logits_ref, topw_ref, topi_ref):
    x = x_ref[...]
    logits = lax.dot_general(x, gw_ref[...], (((1,), (1,)), ((), ())),
                             preferred_element_type=jnp.float32) + gb_ref[...]
    logits_ref[...] = logits
    cur = logits
    iota = lax.broadcasted_iota(jnp.int32, (BT, E), 1)
    ws, ids = [], []
    for _ in range(K):
        m = jnp.max(cur, axis=1, keepdims=True)
        idx = jnp.min(jnp.where(cur == m, iota, E), axis=1, keepdims=True)
        ws.append(m)
        ids.append(idx)
        cur = jnp.where(iota == idx, -1e30, cur)
    l8 = jnp.concatenate(ws, axis=1)
    e8 = jnp.exp(l8 - l8[:, 0:1])
    topw_ref[...] = e8 / jnp.sum(e8, axis=1, keepdims=True)
    topi_ref[...] = jnp.concatenate(ids, axis=1)


def _router(hs, gate_w, gate_b2):
    return pl.pallas_call(
        _router_body,
        grid=(T // BT,),
        in_specs=[
            pl.BlockSpec((BT, H), lambda i: (i, 0)),
            pl.BlockSpec((E, H), lambda i: (0, 0)),
            pl.BlockSpec((1, E), lambda i: (0, 0)),
        ],
        out_specs=[
            pl.BlockSpec((BT, E), lambda i: (i, 0)),
            pl.BlockSpec((BT, K), lambda i: (i, 0)),
            pl.BlockSpec((BT, K), lambda i: (i, 0)),
        ],
        out_shape=[
            jax.ShapeDtypeStruct((T, E), jnp.float32),
            jax.ShapeDtypeStruct((T, K), jnp.float32),
            jax.ShapeDtypeStruct((T, K), jnp.int32),
        ],
        interpret=_INTERPRET,
    )(hs, gate_w, gate_b2)


def _gmm_body(beid_ref, x_ref, w_ref, b_ref, out_ref):
    del beid_ref
    x16 = x_ref[...].astype(jnp.bfloat16)
    w16 = w_ref[0].astype(jnp.bfloat16)
    acc = lax.dot_general(x16, w16, (((1,), (1,)), ((), ())),
                          preferred_element_type=jnp.float32)
    out_ref[...] = acc + b_ref[0]


def _grouped_matmul(block_eid, xs, expert_w, expert_b):
    grid_spec = pltpu.PrefetchScalarGridSpec(
        num_scalar_prefetch=1,
        grid=(NB,),
        in_specs=[
            pl.BlockSpec((BLK, H), lambda i, beid: (i, 0)),
            pl.BlockSpec((1, H, H), lambda i, beid: (beid[i], 0, 0)),
            pl.BlockSpec((1, 1, H), lambda i, beid: (beid[i], 0, 0)),
        ],
        out_specs=pl.BlockSpec((BLK, H), lambda i, beid: (i, 0)),
    )
    return pl.pallas_call(
        _gmm_body,
        grid_spec=grid_spec,
        out_shape=jax.ShapeDtypeStruct((PADDED, H), jnp.float32),
        interpret=_INTERPRET,
    )(block_eid, xs, expert_w, expert_b)


def _make_sc_mesh():
    return plsc.VectorSubcoreMesh(core_axis_name="c", subcore_axis_name="s",
                                  num_cores=NC, num_subcores=NS)


DCT = 32


def _sc_dispatch(hs, dest_kt):
    @functools.partial(
        pl.kernel,
        out_type=jax.ShapeDtypeStruct((PADDED, H), jnp.float32),
        mesh=_make_sc_mesh(),
        scratch_types=[
            pltpu.VMEM((K, DCT), jnp.int32),
            pltpu.VMEM((DCT, H), jnp.float32),
            pltpu.SemaphoreType.DMA,
        ],
    )
    def k(hs_hbm, dest_hbm, out_hbm, idx_v, tok_v, sem):
        wid = lax.axis_index("s") * NC + lax.axis_index("c")
        tbase = wid * TPW

        def body(i, carry):
            t0 = tbase + i * DCT
            pltpu.sync_copy(hs_hbm.at[pl.ds(t0, DCT)], tok_v)
            for kk in range(K):
                pltpu.sync_copy(dest_hbm.at[pl.ds(kk * T + t0, DCT)],
                                idx_v.at[kk])
            for kk in range(K):
                pltpu.async_copy(tok_v, out_hbm.at[idx_v.at[kk]], sem)
            for kk in range(K):
                pltpu.make_async_copy(tok_v, out_hbm.at[idx_v.at[kk]], sem).wait()
            return carry

        lax.fori_loop(0, TPW // DCT, body, 0)

    return k(hs, dest_kt)


def _sc_combine(out_rows, pos, topw_flat):
    @functools.partial(
        pl.kernel,
        out_type=jax.ShapeDtypeStruct((T, H), jnp.float32),
        mesh=_make_sc_mesh(),
        scratch_types=[
            pltpu.VMEM((CT * K,), jnp.int32),
            pltpu.VMEM((CT * K,), jnp.float32),
            pltpu.VMEM((CT * K, H), jnp.float32),
            pltpu.VMEM((CT, H), jnp.float32),
            pltpu.SemaphoreType.DMA,
        ],
    )
    def k(rows_hbm, pos_hbm, w_hbm, out_hbm, idx_v, w_v, rows_v, acc_v, sem):
        wid = lax.axis_index("s") * NC + lax.axis_index("c")
        tbase = wid * TPW

        def body(i, carry):
            t0 = tbase + i * CT
            pltpu.sync_copy(pos_hbm.at[pl.ds(t0 * K, CT * K)], idx_v)
            pltpu.sync_copy(w_hbm.at[pl.ds(t0 * K, CT * K)], w_v)
            pltpu.async_copy(rows_hbm.at[idx_v], rows_v, sem).wait()
            dn = lax.GatherDimensionNumbers(
                offset_dims=(), collapsed_slice_dims=(0,), start_index_map=(0,))
            wsp = []
            for g in range(CT * K // 16):
                wg = w_v[pl.ds(g * 16, 16)]
                for l in range(16):
                    wsp.append(lax.gather(
                        wg, jnp.full((16, 1), l, jnp.int32), dn, (1,),
                        mode=lax.GatherScatterMode.PROMISE_IN_BOUNDS))

            def jbody(j, c2):
                jj = pl.ds(pl.multiple_of(j * 16, 16), 16)
                for t in range(CT):
                    acc = wsp[t * K] * rows_v[t * K, jj]
                    for r in range(1, K):
                        acc = acc + wsp[t * K + r] * rows_v[t * K + r, jj]
                    acc_v[t, jj] = acc
                return c2

            lax.fori_loop(0, H // 16, jbody, 0)
            pltpu.sync_copy(acc_v, out_hbm.at[pl.ds(t0, CT)])
            return carry

        lax.fori_loop(0, TPW // CT, body, 0)

    return k(out_rows, pos, topw_flat)


def kernel(x, gate_w, gate_b, expert_w, expert_b):
    hs = x.reshape(T, H)
    logits, topw, topi = _router(hs, gate_w, gate_b.reshape(1, E))

    onehot = (topi[:, :, None] == jnp.arange(E, dtype=jnp.int32)[None, None, :]
              ).astype(jnp.float32)
    cnt = onehot.sum(axis=1).astype(jnp.int32)
    csum = jnp.cumsum(cnt, axis=0)
    counts = csum[-1]
    cexcl = (csum - cnt).astype(jnp.float32)
    rank = jnp.einsum("tke,te->tk", onehot, cexcl,
                      precision=lax.Precision.HIGHEST)

    padded_counts = ((counts + BLK - 1) // BLK) * BLK
    padded_end = jnp.cumsum(padded_counts).astype(jnp.int32)
    padded_off = (padded_end - padded_counts).astype(jnp.float32)
    off_tk = jnp.einsum("tke,e->tk", onehot, padded_off,
                        precision=lax.Precision.HIGHEST)
    dest = (rank + off_tk).astype(jnp.int32)

    block_eid = jnp.searchsorted(
        padded_end, jnp.arange(NB, dtype=jnp.int32) * BLK, side="right")
    block_eid = jnp.minimum(block_eid, E - 1).astype(jnp.int32)

    xs = _sc_dispatch(hs, dest.T.reshape(P))
    out_rows = _grouped_matmul(block_eid, xs, expert_w,
                               expert_b.reshape(E, 1, H))
    final = _sc_combine(out_rows, dest.reshape(P), topw.reshape(P))
    return final.reshape(B, S, H), logits


# speedup vs baseline: 1.8250x; 1.0397x over previous
import functools

import jax
import jax.numpy as jnp
from jax import lax
from jax.experimental import pallas as pl
from jax.experimental.pallas import tpu as pltpu
from jax.experimental.pallas import tpu_sc as plsc

B, S, H = 4, 8192, 768
E, K = 64, 8
T = B * S
P = T * K
BLK = 256
NB = P // BLK + E
PADDED = NB * BLK
BT = 512

NC, NS = 2, 16
NW = NC * NS
ROWS_PER_W = PADDED // NW
GCH = 64
TPW = T // NW
CT = 8

_INTERPRET = False


def _router_body(x_ref, gw_ref, gb_ref, logits_ref, topw_ref, topi_ref):
    x = x_ref[...]
    logits = lax.dot_general(x, gw_ref[...], (((1,), (1,)), ((), ())),
                             preferred_element_type=jnp.float32) + gb_ref[...]
    logits_ref[...] = logits
    cur = logits
    iota = lax.broadcasted_iota(jnp.int32, (BT, E), 1)
    ws, ids = [], []
    for _ in range(K):
        m = jnp.max(cur, axis=1, keepdims=True)
        idx = jnp.min(jnp.where(cur == m, iota, E), axis=1, keepdims=True)
        ws.append(m)
        ids.append(idx)
        cur = jnp.where(iota == idx, -1e30, cur)
    l8 = jnp.concatenate(ws, axis=1)
    e8 = jnp.exp(l8 - l8[:, 0:1])
    topw_ref[...] = e8 / jnp.sum(e8, axis=1, keepdims=True)
    topi_ref[...] = jnp.concatenate(ids, axis=1)


def _router(hs, gate_w, gate_b2):
    return pl.pallas_call(
        _router_body,
        grid=(T // BT,),
        in_specs=[
            pl.BlockSpec((BT, H), lambda i: (i, 0)),
            pl.BlockSpec((E, H), lambda i: (0, 0)),
            pl.BlockSpec((1, E), lambda i: (0, 0)),
        ],
        out_specs=[
            pl.BlockSpec((BT, E), lambda i: (i, 0)),
            pl.BlockSpec((BT, K), lambda i: (i, 0)),
            pl.BlockSpec((BT, K), lambda i: (i, 0)),
        ],
        out_shape=[
            jax.ShapeDtypeStruct((T, E), jnp.float32),
            jax.ShapeDtypeStruct((T, K), jnp.float32),
            jax.ShapeDtypeStruct((T, K), jnp.int32),
        ],
        interpret=_INTERPRET,
    )(hs, gate_w, gate_b2)


def _gmm_body(beid_ref, x_ref, w_ref, b_ref, wp_ref, out_ref):
    del beid_ref
    x16 = x_ref[...].astype(jnp.bfloat16)
    w16 = w_ref[0].astype(jnp.bfloat16)
    acc = lax.dot_general(x16, w16, (((1,), (1,)), ((), ())),
                          preferred_element_type=jnp.float32)
    out_ref[...] = (acc + b_ref[0]) * wp_ref[...]


def _grouped_matmul(block_eid, xs, expert_w, expert_b, wp):
    grid_spec = pltpu.PrefetchScalarGridSpec(
        num_scalar_prefetch=1,
        grid=(NB,),
        in_specs=[
            pl.BlockSpec((BLK, H), lambda i, beid: (i, 0)),
            pl.BlockSpec((1, H, H), lambda i, beid: (beid[i], 0, 0)),
            pl.BlockSpec((1, 1, H), lambda i, beid: (beid[i], 0, 0)),
            pl.BlockSpec((BLK, 1), lambda i, beid: (i, 0)),
        ],
        out_specs=pl.BlockSpec((BLK, H), lambda i, beid: (i, 0)),
    )
    return pl.pallas_call(
        _gmm_body,
        grid_spec=grid_spec,
        out_shape=jax.ShapeDtypeStruct((PADDED, H), jnp.float32),
        interpret=_INTERPRET,
    )(block_eid, xs, expert_w, expert_b, wp)


def _make_sc_mesh():
    return plsc.VectorSubcoreMesh(core_axis_name="c", subcore_axis_name="s",
                                  num_cores=NC, num_subcores=NS)


DCT = 32


def _sc_dispatch(hs, dest_kt, w_kt):
    @functools.partial(
        pl.kernel,
        out_type=(jax.ShapeDtypeStruct((PADDED, H), jnp.float32),
                  jax.ShapeDtypeStruct((PADDED,), jnp.float32)),
        mesh=_make_sc_mesh(),
        scratch_types=[
            pltpu.VMEM((K, DCT), jnp.int32),
            pltpu.VMEM((K, DCT), jnp.float32),
            pltpu.VMEM((DCT, H), jnp.float32),
            pltpu.SemaphoreType.DMA,
        ],
    )
    def k(hs_hbm, dest_hbm, w_hbm, out_hbm, ws_hbm, idx_v, wv, tok_v, sem):
        wid = lax.axis_index("s") * NC + lax.axis_index("c")
        tbase = wid * TPW

        def body(i, carry):
            t0 = tbase + i * DCT
            pltpu.sync_copy(hs_hbm.at[pl.ds(t0, DCT)], tok_v)
            for kk in range(K):
                pltpu.sync_copy(dest_hbm.at[pl.ds(kk * T + t0, DCT)],
                                idx_v.at[kk])
                pltpu.sync_copy(w_hbm.at[pl.ds(kk * T + t0, DCT)], wv.at[kk])
            for kk in range(K):
                pltpu.async_copy(tok_v, out_hbm.at[idx_v.at[kk]], sem)
                pltpu.async_copy(wv.at[kk], ws_hbm.at[idx_v.at[kk]], sem)
            for kk in range(K):
                pltpu.make_async_copy(tok_v, out_hbm.at[idx_v.at[kk]], sem).wait()
                pltpu.make_async_copy(wv.at[kk], ws_hbm.at[idx_v.at[kk]], sem).wait()
            return carry

        lax.fori_loop(0, TPW // DCT, body, 0)

    return k(hs, dest_kt, w_kt)


def _sc_combine(out_rows, pos):
    @functools.partial(
        pl.kernel,
        out_type=jax.ShapeDtypeStruct((T, H), jnp.float32),
        mesh=_make_sc_mesh(),
        scratch_types=[
            pltpu.VMEM((CT * K,), jnp.int32),
            pltpu.VMEM((CT * K, H), jnp.float32),
            pltpu.VMEM((CT, H), jnp.float32),
            pltpu.SemaphoreType.DMA,
        ],
    )
    def k(rows_hbm, pos_hbm, out_hbm, idx_v, rows_v, acc_v, sem):
        wid = lax.axis_index("s") * NC + lax.axis_index("c")
        tbase = wid * TPW

        def body(i, carry):
            t0 = tbase + i * CT
            pltpu.sync_copy(pos_hbm.at[pl.ds(t0 * K, CT * K)], idx_v)
            pltpu.async_copy(rows_hbm.at[idx_v], rows_v, sem).wait()

            def jbody(j, c2):
                jj = pl.ds(pl.multiple_of(j * 16, 16), 16)
                for t in range(CT):
                    acc = rows_v[t * K, jj]
                    for r in range(1, K):
                        acc = acc + rows_v[t * K + r, jj]
                    acc_v[t, jj] = acc
                return c2

            lax.fori_loop(0, H // 16, jbody, 0)
            pltpu.sync_copy(acc_v, out_hbm.at[pl.ds(t0, CT)])
            return carry

        lax.fori_loop(0, TPW // CT, body, 0)

    return k(out_rows, pos)


def kernel(x, gate_w, gate_b, expert_w, expert_b):
    hs = x.reshape(T, H)
    logits, topw, topi = _router(hs, gate_w, gate_b.reshape(1, E))

    onehot = (topi[:, :, None] == jnp.arange(E, dtype=jnp.int32)[None, None, :]
              ).astype(jnp.float32)
    cnt = onehot.sum(axis=1).astype(jnp.int32)
    csum = jnp.cumsum(cnt, axis=0)
    counts = csum[-1]
    cexcl = (csum - cnt).astype(jnp.float32)
    rank = jnp.einsum("tke,te->tk", onehot, cexcl,
                      precision=lax.Precision.HIGHEST)

    padded_counts = ((counts + BLK - 1) // BLK) * BLK
    padded_end = jnp.cumsum(padded_counts).astype(jnp.int32)
    padded_off = (padded_end - padded_counts).astype(jnp.float32)
    off_tk = jnp.einsum("tke,e->tk", onehot, padded_off,
                        precision=lax.Precision.HIGHEST)
    dest = (rank + off_tk).astype(jnp.int32)

    block_eid = jnp.searchsorted(
        padded_end, jnp.arange(NB, dtype=jnp.int32) * BLK, side="right")
    block_eid = jnp.minimum(block_eid, E - 1).astype(jnp.int32)

    xs, ws = _sc_dispatch(hs, dest.T.reshape(P), topw.T.reshape(P))
    out_rows = _grouped_matmul(block_eid, xs, expert_w,
                               expert_b.reshape(E, 1, H),
                               ws.reshape(PADDED, 1))
    final = _sc_combine(out_rows, dest.reshape(P))
    return final.reshape(B, S, H), logits
